# baseline (speedup 1.0000x reference)
import functools

import jax
import jax.numpy as jnp
from jax import lax
from jax.experimental import pallas as pl
from jax.experimental.pallas import tpu as pltpu
from jax.experimental.pallas import tpu_sc as plsc

_B = 16384
_V = 100000
_D = 16
_H = 128
_C = 387
_NC = 2
_NS = 16
_CHUNK = 4096

_F32 = jnp.float32


def _serve_column(tbl, idx_hbm, col, row_off, base, n_rows,
                  col_v, idx_v, res_v, sem_col, sem_idx, sem_out, out):
    cp_idx = pltpu.async_copy(idx_hbm.at[pl.ds(base, n_rows)],
                              idx_v.at[pl.ds(0, n_rows)], sem_idx)
    cp_col = pltpu.async_copy(tbl.at[col], col_v, sem_col)
    cp_idx.wait()
    cp_col.wait()
    out_cps = []
    for chunk in range(n_rows // _CHUNK):
        half = (chunk % 2) * _CHUNK
        if chunk >= 2:
            out_cps[chunk - 2].wait()

        @pl.loop(0, _CHUNK // 128)
        def _gather(i):
            base_w = chunk * _CHUNK + i * 128
            res_w = half + i * 128
            ids = [idx_v[pl.ds(base_w + j * 16, 16)] for j in range(8)]
            vals = [plsc.load_gather(col_v, [v]) for v in ids]
            for j, v in enumerate(vals):
                res_v[pl.ds(res_w + j * 16, 16)] = v

        out_cps.append(pltpu.async_copy(
            res_v.at[pl.ds(half, _CHUNK)],
            out.at[row_off + col, pl.ds(base + chunk * _CHUNK, _CHUNK)],
            sem_out))
    for cp in out_cps[-2:]:
        cp.wait()


def _sc_gather_body(m_idx, n_idx, k_idx, tbl_m, tbl_n, tbl_k, out,
                    col_v, idx_v, res_v, sem_col, sem_idx, sem_out):
    c = lax.axis_index("c")
    s = lax.axis_index("s")
    sems = (sem_col, sem_idx, sem_out)

    @pl.when(c == 0)
    def _():
        _serve_column(tbl_m, m_idx, s, 0, 0, _B, col_v, idx_v, res_v,
                      *sems, out)

    @pl.when(c == 1)
    def _():
        _serve_column(tbl_n, n_idx, s, _D, 0, _B, col_v, idx_v, res_v,
                      *sems, out)

    @pl.when(s < _NS // 2)
    def _():
        _serve_column(tbl_k, k_idx, c * (_NS // 2) + s, 2 * _D, 0, _B,
                      col_v, idx_v, res_v, *sems, out)


def _sc_gather(m_i, n_i, k_i, tbl_m_t, tbl_n_t, tbl_k_t):
    f = pl.kernel(
        _sc_gather_body,
        out_type=jax.ShapeDtypeStruct((3 * _D, _B), _F32),
        mesh=plsc.VectorSubcoreMesh(core_axis_name="c", subcore_axis_name="s"),
        compiler_params=pltpu.CompilerParams(needs_layout_passes=False),
        scratch_types=[
            pltpu.VMEM((_V,), _F32),
            pltpu.VMEM((_B,), jnp.int32),
            pltpu.VMEM((2 * _CHUNK,), _F32),
            pltpu.SemaphoreType.DMA,
            pltpu.SemaphoreType.DMA,
            pltpu.SemaphoreType.DMA,
        ],
    )
    return f(m_i, n_i, k_i, tbl_m_t, tbl_n_t, tbl_k_t)


_BB = 2048


def _mlp_body(cat_ref, w1t_ref, b1_ref, w2t_ref, b2_ref, out_ref):
    et = cat_ref[...]
    ht = jnp.dot(w1t_ref[...], et, preferred_element_type=_F32)
    ht = jnp.maximum(ht + b1_ref[...], 0.0)
    lt = jnp.dot(w2t_ref[...], ht, preferred_element_type=_F32)
    lt = lt + b2_ref[...]
    m = jnp.max(lt, axis=0, keepdims=True)
    e = jnp.exp(lt - m)
    out_ref[...] = e / jnp.sum(e, axis=0, keepdims=True)


def _mlp_t(cat_t, w1, b1, w2, b2):
    w1t = w1.T
    w2t = w2.T
    b1r = b1.reshape(_H, 1)
    b2r = b2.reshape(_C, 1)
    grid = (_B // _BB,)
    full = lambda shape: pl.BlockSpec(shape, lambda i: (0, 0))
    return pl.pallas_call(
        _mlp_body,
        grid=grid,
        in_specs=[
            pl.BlockSpec((3 * _D, _BB), lambda i: (0, i)),
            full((_H, 3 * _D)), full((_H, 1)),
            full((_C, _H)), full((_C, 1)),
        ],
        out_specs=pl.BlockSpec((_C, _BB), lambda i: (0, i)),
        out_shape=jax.ShapeDtypeStruct((_C, _B), _F32),
    )(cat_t, w1t, b1r, w2t, b2r)


def kernel(M, N, K, emb_M, emb_N, emb_K, W1, b1, W2, b2):
    m_i = M.astype(jnp.int32)
    n_i = N.astype(jnp.int32)
    k_i = K.astype(jnp.int32)
    cat_t = _sc_gather(m_i, n_i, k_i, emb_M.T, emb_N.T, emb_K.T)
    out_t = _mlp_t(cat_t, W1, b1, W2, b2)
    return out_t.T


# speedup vs baseline: 1.0198x; 1.0198x over previous
import functools

import jax
import jax.numpy as jnp
from jax import lax
from jax.experimental import pallas as pl
from jax.experimental.pallas import tpu as pltpu
from jax.experimental.pallas import tpu_sc as plsc

_B = 16384
_V = 100000
_D = 16
_H = 128
_C = 387
_NC = 2
_NS = 16
_CHUNK = 4096

_F32 = jnp.float32


def _serve_column(tbl, idx_hbm, col, row_off, base, n_rows,
                  col_v, idx_v, res_v, sem_col, sem_idx, sem_out, out):
    cp_idx = pltpu.async_copy(idx_hbm.at[pl.ds(base, n_rows)],
                              idx_v.at[pl.ds(0, n_rows)], sem_idx)
    cp_col = pltpu.async_copy(tbl.at[col], col_v, sem_col)
    cp_idx.wait()
    cp_col.wait()
    out_cps = []
    for chunk in range(n_rows // _CHUNK):
        half = (chunk % 2) * _CHUNK
        if chunk >= 2:
            out_cps[chunk - 2].wait()

        @pl.loop(0, _CHUNK // 128)
        def _gather(i):
            base_w = chunk * _CHUNK + i * 128
            res_w = half + i * 128
            ids = [idx_v[pl.ds(base_w + j * 16, 16)] for j in range(8)]
            vals = [plsc.load_gather(col_v, [v]) for v in ids]
            for j, v in enumerate(vals):
                res_v[pl.ds(res_w + j * 16, 16)] = v

        out_cps.append(pltpu.async_copy(
            res_v.at[pl.ds(half, _CHUNK)],
            out.at[row_off + col, pl.ds(base + chunk * _CHUNK, _CHUNK)],
            sem_out))
    for cp in out_cps[-2:]:
        cp.wait()


def _sc_gather_body(m_idx, n_idx, k_idx, tbl_m, tbl_n, tbl_k, out,
                    col_v, idx_v, res_v, sem_col, sem_idx, sem_out):
    c = lax.axis_index("c")
    s = lax.axis_index("s")
    sems = (sem_col, sem_idx, sem_out)

    @pl.when(c == 0)
    def _():
        _serve_column(tbl_m, m_idx, s, 0, 0, _B, col_v, idx_v, res_v,
                      *sems, out)

    @pl.when(c == 1)
    def _():
        _serve_column(tbl_n, n_idx, s, _D, 0, _B, col_v, idx_v, res_v,
                      *sems, out)

    _serve_column(tbl_k, k_idx, s, 2 * _D, c * (_B // 2), _B // 2,
                  col_v, idx_v, res_v, *sems, out)


def _sc_gather(m_i, n_i, k_i, tbl_m_t, tbl_n_t, tbl_k_t):
    f = pl.kernel(
        _sc_gather_body,
        out_type=jax.ShapeDtypeStruct((3 * _D, _B), _F32),
        mesh=plsc.VectorSubcoreMesh(core_axis_name="c", subcore_axis_name="s"),
        compiler_params=pltpu.CompilerParams(needs_layout_passes=False),
        scratch_types=[
            pltpu.VMEM((_V,), _F32),
            pltpu.VMEM((_B,), jnp.int32),
            pltpu.VMEM((2 * _CHUNK,), _F32),
            pltpu.SemaphoreType.DMA,
            pltpu.SemaphoreType.DMA,
            pltpu.SemaphoreType.DMA,
        ],
    )
    return f(m_i, n_i, k_i, tbl_m_t, tbl_n_t, tbl_k_t)


_BB = 4096


def _mlp_body(cat_ref, w1t_ref, b1_ref, w2t_ref, b2_ref, out_ref):
    et = cat_ref[...]
    ht = jnp.dot(w1t_ref[...], et, preferred_element_type=_F32)
    ht = jnp.maximum(ht + b1_ref[...], 0.0)
    lt = jnp.dot(w2t_ref[...], ht, preferred_element_type=_F32)
    lt = lt + b2_ref[...]
    m = jnp.max(lt, axis=0, keepdims=True)
    e = jnp.exp(lt - m)
    out_ref[...] = e / jnp.sum(e, axis=0, keepdims=True)


def _mlp_t(cat_t, w1, b1, w2, b2):
    w1t = w1.T
    w2t = w2.T
    b1r = b1.reshape(_H, 1)
    b2r = b2.reshape(_C, 1)
    grid = (_B // _BB,)
    full = lambda shape: pl.BlockSpec(shape, lambda i: (0, 0))
    return pl.pallas_call(
        _mlp_body,
        grid=grid,
        in_specs=[
            pl.BlockSpec((3 * _D, _BB), lambda i: (0, i)),
            full((_H, 3 * _D)), full((_H, 1)),
            full((_C, _H)), full((_C, 1)),
        ],
        out_specs=pl.BlockSpec((_C, _BB), lambda i: (0, i)),
        out_shape=jax.ShapeDtypeStruct((_C, _B), _F32),
    )(cat_t, w1t, b1r, w2t, b2r)


def kernel(M, N, K, emb_M, emb_N, emb_K, W1, b1, W2, b2):
    m_i = M.astype(jnp.int32)
    n_i = N.astype(jnp.int32)
    k_i = K.astype(jnp.int32)
    cat_t = _sc_gather(m_i, n_i, k_i, emb_M.T, emb_N.T, emb_K.T)
    out_t = _mlp_t(cat_t, W1, b1, W2, b2)
    return out_t.T
